# baseline (speedup 1.0000x reference)
import functools

import jax
import jax.numpy as jnp
from jax import lax
from jax.experimental import pallas as pl
from jax.experimental.pallas import tpu as pltpu
from jax.experimental.pallas import tpu_sc as plsc

_TB = 512
_NCHUNK = 8


def _affinity_block(x_ref, w_ref, aff_ref):
    x = x_ref[...]
    logits = jnp.dot(x, w_ref[...], preferred_element_type=jnp.float32)
    sp = jnp.maximum(logits, 0.0) + jnp.log1p(jnp.exp(-jnp.abs(logits)))
    aff_ref[...] = jnp.sqrt(sp + 1e-12)


def _affinity_call(x, W, chunk, n_chunks):
    T, D = x.shape
    E = W.shape[1]
    tb = _TB
    tc = T // n_chunks
    steps = tc // tb
    return pl.pallas_call(
        _affinity_block,
        grid=(steps,),
        in_specs=[
            pl.BlockSpec((tb, D), lambda i, c=chunk, s=steps: (c * s + i, 0)),
            pl.BlockSpec((D, E), lambda i: (0, 0)),
        ],
        out_specs=pl.BlockSpec((tb, E), lambda i: (i, 0)),
        out_shape=jax.ShapeDtypeStruct((tc, E), jnp.float32),
    )(x, W)


def _sc_topk_body(aff_hbm, vis_hbm, aux_hbm, mb_hbm, idx_hbm, gate_hbm,
                  aff_v, vis_v, aux_v, mb_v, idxo_v, gateo_v, *, n_tok):
    wid = lax.axis_index("s") * 2 + lax.axis_index("c")
    base = wid * n_tok

    pltpu.sync_copy(aff_hbm.at[pl.ds(base, n_tok), :], aff_v)
    pltpu.sync_copy(vis_hbm.at[pl.ds(base, n_tok)], vis_v)
    pltpu.sync_copy(aux_hbm, aux_v)
    pltpu.sync_copy(mb_hbm, mb_v)

    ii = lax.broadcasted_iota(jnp.int32, (16,), 0)
    lo8 = ii < 8
    shifted = jnp.maximum(ii - 8, 0)

    pre = []
    for c in range(4):
        a_c = aux_v[pl.ds(c * 16, 16)]
        pre.append((a_c + mb_v[0, pl.ds(c * 16, 16)],
                    a_c + mb_v[1, pl.ds(c * 16, 16)]))

    def merge_top8(ak, av, bk, bv):
        ck = jnp.where(lo8, ak, bk.at[shifted].get(mode="promise_in_bounds"))
        cv = jnp.where(lo8, av, bv.at[shifted].get(mode="promise_in_bounds"))
        return plsc.sort_key_val(ck, cv, descending=True)

    def one_token(t):
        t_splat = jnp.full((16,), t, jnp.int32)
        vis_t = plsc.load_gather(vis_v, [t_splat])
        visb = vis_t != 0
        sk, sv = [], []
        for c in range(4):
            b_c = aff_v[t, pl.ds(c * 16, 16)] + jnp.where(visb, pre[c][1], pre[c][0])
            kk, vv = plsc.sort_key_val(b_c, ii + 16 * c, descending=True)
            sk.append(kk)
            sv.append(vv)
        mk0, mv0 = merge_top8(sk[0], sv[0], sk[1], sv[1])
        mk1, mv1 = merge_top8(sk[2], sv[2], sk[3], sv[3])
        _, fv = merge_top8(mk0, mv0, mk1, mv1)

        g_aff = plsc.load_gather(aff_v, [t_splat, fv], mask=lo8)
        g_aff = jnp.where(lo8, g_aff, 0.0)
        gate8 = g_aff / (jnp.sum(g_aff, axis=0) + 1e-12)

        opos = t * 8 + ii
        plsc.store_scatter(idxo_v, [opos], fv, mask=lo8)
        plsc.store_scatter(gateo_v, [opos], gate8, mask=lo8)

    def tok_body(i, _):
        one_token(2 * i)
        one_token(2 * i + 1)
        return _

    lax.fori_loop(0, n_tok // 2, tok_body, 0)

    pltpu.sync_copy(idxo_v, idx_hbm.at[pl.ds(base * 8, n_tok * 8)])
    pltpu.sync_copy(gateo_v, gate_hbm.at[pl.ds(base * 8, n_tok * 8)])


def _sc_topk_call(aff, visi, aux, mb):
    Tc, E = aff.shape
    n_tok = Tc // 32
    body = functools.partial(_sc_topk_body, n_tok=n_tok)
    fn = pl.kernel(
        body,
        out_type=[
            jax.ShapeDtypeStruct((Tc * 8,), jnp.int32),
            jax.ShapeDtypeStruct((Tc * 8,), jnp.float32),
        ],
        mesh=plsc.VectorSubcoreMesh(core_axis_name="c", subcore_axis_name="s"),
        compiler_params=pltpu.CompilerParams(needs_layout_passes=False),
        scratch_types=[
            pltpu.VMEM((n_tok, E), jnp.float32),
            pltpu.VMEM((n_tok,), jnp.int32),
            pltpu.VMEM((E,), jnp.float32),
            pltpu.VMEM((2, E), jnp.float32),
            pltpu.VMEM((n_tok * 8,), jnp.int32),
            pltpu.VMEM((n_tok * 8,), jnp.float32),
        ],
    )
    return fn(aff, visi, aux, mb)


def kernel(x, is_visual, W, aux_free_bias, modality_bias):
    T, D = x.shape
    E = W.shape[1]
    nc = _NCHUNK
    tc = T // nc
    visi = is_visual.astype(jnp.int32)

    affs, idxs, gates = [], [], []
    for c in range(nc):
        aff_c = _affinity_call(x, W, c, nc)
        idxf, gatef = _sc_topk_call(aff_c, visi[c * tc:(c + 1) * tc],
                                    aux_free_bias, modality_bias)
        affs.append(aff_c)
        idxs.append(idxf.reshape(tc, 8))
        gates.append(gatef.reshape(tc, 8))

    return (jnp.concatenate(idxs, axis=0),
            jnp.concatenate(gates, axis=0),
            jnp.concatenate(affs, axis=0))


# speedup vs baseline: 1.0356x; 1.0356x over previous
import functools

import jax
import jax.numpy as jnp
from jax import lax
from jax.experimental import pallas as pl
from jax.experimental.pallas import tpu as pltpu
from jax.experimental.pallas import tpu_sc as plsc

_TB = 512
_NCHUNK = 2


def _affinity_block(x_ref, w_ref, aff_ref):
    x = x_ref[...]
    logits = jnp.dot(x, w_ref[...], preferred_element_type=jnp.float32)
    sp = jnp.maximum(logits, 0.0) + jnp.log1p(jnp.exp(-jnp.abs(logits)))
    aff_ref[...] = jnp.sqrt(sp + 1e-12)


def _affinity_call(x, W, chunk, n_chunks):
    T, D = x.shape
    E = W.shape[1]
    tb = _TB
    tc = T // n_chunks
    steps = tc // tb
    return pl.pallas_call(
        _affinity_block,
        grid=(steps,),
        in_specs=[
            pl.BlockSpec((tb, D), lambda i, c=chunk, s=steps: (c * s + i, 0)),
            pl.BlockSpec((D, E), lambda i: (0, 0)),
        ],
        out_specs=pl.BlockSpec((tb, E), lambda i: (i, 0)),
        out_shape=jax.ShapeDtypeStruct((tc, E), jnp.float32),
    )(x, W)


def _sc_topk_body(aff_hbm, vis_hbm, aux_hbm, mb_hbm, idx_hbm, gate_hbm,
                  aff_v, vis_v, aux_v, mb_v, idxo_v, gateo_v, *, n_tok):
    wid = lax.axis_index("s") * 2 + lax.axis_index("c")
    base = wid * n_tok

    pltpu.sync_copy(aff_hbm.at[pl.ds(base, n_tok), :], aff_v)
    pltpu.sync_copy(vis_hbm.at[pl.ds(base, n_tok)], vis_v)
    pltpu.sync_copy(aux_hbm, aux_v)
    pltpu.sync_copy(mb_hbm, mb_v)

    ii = lax.broadcasted_iota(jnp.int32, (16,), 0)
    lo8 = ii < 8
    shifted = jnp.maximum(ii - 8, 0)

    pre = []
    for c in range(4):
        a_c = aux_v[pl.ds(c * 16, 16)]
        pre.append((a_c + mb_v[0, pl.ds(c * 16, 16)],
                    a_c + mb_v[1, pl.ds(c * 16, 16)]))

    def merge_top8(ak, av, bk, bv):
        ck = jnp.where(lo8, ak, bk.at[shifted].get(mode="promise_in_bounds"))
        cv = jnp.where(lo8, av, bv.at[shifted].get(mode="promise_in_bounds"))
        return plsc.sort_key_val(ck, cv, descending=True)

    def one_token(t):
        t_splat = jnp.full((16,), t, jnp.int32)
        vis_t = plsc.load_gather(vis_v, [t_splat])
        visb = vis_t != 0
        sk, sv = [], []
        for c in range(4):
            b_c = aff_v[t, pl.ds(c * 16, 16)] + jnp.where(visb, pre[c][1], pre[c][0])
            kk, vv = plsc.sort_key_val(b_c, ii + 16 * c, descending=True)
            sk.append(kk)
            sv.append(vv)
        mk0, mv0 = merge_top8(sk[0], sv[0], sk[1], sv[1])
        mk1, mv1 = merge_top8(sk[2], sv[2], sk[3], sv[3])
        _, fv = merge_top8(mk0, mv0, mk1, mv1)

        g_aff = plsc.load_gather(aff_v, [t_splat, fv], mask=lo8)
        g_aff = jnp.where(lo8, g_aff, 0.0)
        gate8 = g_aff / (jnp.sum(g_aff, axis=0) + 1e-12)

        opos = t * 8 + ii
        plsc.store_scatter(idxo_v, [opos], fv, mask=lo8)
        plsc.store_scatter(gateo_v, [opos], gate8, mask=lo8)

    def tok_body(i, _):
        one_token(2 * i)
        one_token(2 * i + 1)
        return _

    lax.fori_loop(0, n_tok // 2, tok_body, 0)

    pltpu.sync_copy(idxo_v, idx_hbm.at[pl.ds(base * 8, n_tok * 8)])
    pltpu.sync_copy(gateo_v, gate_hbm.at[pl.ds(base * 8, n_tok * 8)])


def _sc_topk_call(aff, visi, aux, mb):
    Tc, E = aff.shape
    n_tok = Tc // 32
    body = functools.partial(_sc_topk_body, n_tok=n_tok)
    fn = pl.kernel(
        body,
        out_type=[
            jax.ShapeDtypeStruct((Tc * 8,), jnp.int32),
            jax.ShapeDtypeStruct((Tc * 8,), jnp.float32),
        ],
        mesh=plsc.VectorSubcoreMesh(core_axis_name="c", subcore_axis_name="s"),
        compiler_params=pltpu.CompilerParams(needs_layout_passes=False),
        scratch_types=[
            pltpu.VMEM((n_tok, E), jnp.float32),
            pltpu.VMEM((n_tok,), jnp.int32),
            pltpu.VMEM((E,), jnp.float32),
            pltpu.VMEM((2, E), jnp.float32),
            pltpu.VMEM((n_tok * 8,), jnp.int32),
            pltpu.VMEM((n_tok * 8,), jnp.float32),
        ],
    )
    return fn(aff, visi, aux, mb)


def kernel(x, is_visual, W, aux_free_bias, modality_bias):
    T, D = x.shape
    E = W.shape[1]
    nc = _NCHUNK
    tc = T // nc
    visi = is_visual.astype(jnp.int32)

    affs, idxs, gates = [], [], []
    for c in range(nc):
        aff_c = _affinity_call(x, W, c, nc)
        idxf, gatef = _sc_topk_call(aff_c, visi[c * tc:(c + 1) * tc],
                                    aux_free_bias, modality_bias)
        affs.append(aff_c)
        idxs.append(idxf.reshape(tc, 8))
        gates.append(gatef.reshape(tc, 8))

    return (jnp.concatenate(idxs, axis=0),
            jnp.concatenate(gates, axis=0),
            jnp.concatenate(affs, axis=0))


# speedup vs baseline: 1.0686x; 1.0318x over previous
import functools

import jax
import jax.numpy as jnp
from jax import lax
from jax.experimental import pallas as pl
from jax.experimental.pallas import tpu as pltpu
from jax.experimental.pallas import tpu_sc as plsc

_TB = 512
_CHUNKS = (10240, 10240, 10240, 2048)


def _affinity_block(x_ref, w_ref, aff_ref):
    x = x_ref[...]
    logits = jnp.dot(x, w_ref[...], preferred_element_type=jnp.float32)
    sp = jnp.maximum(logits, 0.0) + jnp.log1p(jnp.exp(-jnp.abs(logits)))
    aff_ref[...] = jnp.sqrt(sp + 1e-12)


def _affinity_call(x, W, start, size):
    T, D = x.shape
    E = W.shape[1]
    tb = _TB
    steps = size // tb
    base = start // tb
    return pl.pallas_call(
        _affinity_block,
        grid=(steps,),
        in_specs=[
            pl.BlockSpec((tb, D), lambda i, b=base: (b + i, 0)),
            pl.BlockSpec((D, E), lambda i: (0, 0)),
        ],
        out_specs=pl.BlockSpec((tb, E), lambda i: (i, 0)),
        out_shape=jax.ShapeDtypeStruct((size, E), jnp.float32),
    )(x, W)


def _sc_topk_body(aff_hbm, vis_hbm, aux_hbm, mb_hbm, idx_hbm, gate_hbm,
                  aff_v, vis_v, aux_v, mb_v, idxo_v, gateo_v, *, n_tok):
    wid = lax.axis_index("s") * 2 + lax.axis_index("c")
    base = wid * n_tok

    pltpu.sync_copy(aff_hbm.at[pl.ds(base, n_tok), :], aff_v)
    pltpu.sync_copy(vis_hbm.at[pl.ds(base, n_tok)], vis_v)
    pltpu.sync_copy(aux_hbm, aux_v)
    pltpu.sync_copy(mb_hbm, mb_v)

    ii = lax.broadcasted_iota(jnp.int32, (16,), 0)
    lo8 = ii < 8
    shifted = jnp.maximum(ii - 8, 0)

    pre = []
    for c in range(4):
        a_c = aux_v[pl.ds(c * 16, 16)]
        pre.append((a_c + mb_v[0, pl.ds(c * 16, 16)],
                    a_c + mb_v[1, pl.ds(c * 16, 16)]))

    def merge_top8(ak, av, bk, bv):
        ck = jnp.where(lo8, ak, bk.at[shifted].get(mode="promise_in_bounds"))
        cv = jnp.where(lo8, av, bv.at[shifted].get(mode="promise_in_bounds"))
        return plsc.sort_key_val(ck, cv, descending=True)

    def one_token(t):
        t_splat = jnp.full((16,), t, jnp.int32)
        vis_t = plsc.load_gather(vis_v, [t_splat])
        visb = vis_t != 0
        sk, sv = [], []
        for c in range(4):
            b_c = aff_v[t, pl.ds(c * 16, 16)] + jnp.where(visb, pre[c][1], pre[c][0])
            kk, vv = plsc.sort_key_val(b_c, ii + 16 * c, descending=True)
            sk.append(kk)
            sv.append(vv)
        mk0, mv0 = merge_top8(sk[0], sv[0], sk[1], sv[1])
        mk1, mv1 = merge_top8(sk[2], sv[2], sk[3], sv[3])
        _, fv = merge_top8(mk0, mv0, mk1, mv1)

        g_aff = plsc.load_gather(aff_v, [t_splat, fv], mask=lo8)
        g_aff = jnp.where(lo8, g_aff, 0.0)
        gate8 = g_aff / (jnp.sum(g_aff, axis=0) + 1e-12)

        opos = t * 8 + ii
        plsc.store_scatter(idxo_v, [opos], fv, mask=lo8)
        plsc.store_scatter(gateo_v, [opos], gate8, mask=lo8)

    def tok_body(i, _):
        one_token(2 * i)
        one_token(2 * i + 1)
        return _

    lax.fori_loop(0, n_tok // 2, tok_body, 0)

    pltpu.sync_copy(idxo_v, idx_hbm.at[pl.ds(base * 8, n_tok * 8)])
    pltpu.sync_copy(gateo_v, gate_hbm.at[pl.ds(base * 8, n_tok * 8)])


def _sc_topk_call(aff, visi, aux, mb):
    Tc, E = aff.shape
    n_tok = Tc // 32
    body = functools.partial(_sc_topk_body, n_tok=n_tok)
    fn = pl.kernel(
        body,
        out_type=[
            jax.ShapeDtypeStruct((Tc * 8,), jnp.int32),
            jax.ShapeDtypeStruct((Tc * 8,), jnp.float32),
        ],
        mesh=plsc.VectorSubcoreMesh(core_axis_name="c", subcore_axis_name="s"),
        compiler_params=pltpu.CompilerParams(needs_layout_passes=False),
        scratch_types=[
            pltpu.VMEM((n_tok, E), jnp.float32),
            pltpu.VMEM((n_tok,), jnp.int32),
            pltpu.VMEM((E,), jnp.float32),
            pltpu.VMEM((2, E), jnp.float32),
            pltpu.VMEM((n_tok * 8,), jnp.int32),
            pltpu.VMEM((n_tok * 8,), jnp.float32),
        ],
    )
    return fn(aff, visi, aux, mb)


def kernel(x, is_visual, W, aux_free_bias, modality_bias):
    T, D = x.shape
    E = W.shape[1]
    visi = is_visual.astype(jnp.int32)

    affs, idxs, gates = [], [], []
    start = 0
    for tc in _CHUNKS:
        aff_c = _affinity_call(x, W, start, tc)
        idxf, gatef = _sc_topk_call(aff_c, visi[start:start + tc],
                                    aux_free_bias, modality_bias)
        affs.append(aff_c)
        idxs.append(idxf.reshape(tc, 8))
        gates.append(gatef.reshape(tc, 8))
        start += tc

    return (jnp.concatenate(idxs, axis=0),
            jnp.concatenate(gates, axis=0),
            jnp.concatenate(affs, axis=0))


# speedup vs baseline: 1.0801x; 1.0108x over previous
import functools

import jax
import jax.numpy as jnp
from jax import lax
from jax.experimental import pallas as pl
from jax.experimental.pallas import tpu as pltpu
from jax.experimental.pallas import tpu_sc as plsc

_TB = 1024
_NCHUNK = 4


def _affinity_block(x_ref, w_ref, aff_ref):
    x = x_ref[...]
    logits = jnp.dot(x, w_ref[...], preferred_element_type=jnp.float32)
    sp = jnp.maximum(logits, 0.0) + jnp.log1p(jnp.exp(-jnp.abs(logits)))
    aff_ref[...] = jnp.sqrt(sp + 1e-12)


def _affinity_call(x, W, chunk, n_chunks):
    T, D = x.shape
    E = W.shape[1]
    tb = _TB
    tc = T // n_chunks
    steps = tc // tb
    return pl.pallas_call(
        _affinity_block,
        grid=(steps,),
        in_specs=[
            pl.BlockSpec((tb, D), lambda i, c=chunk, s=steps: (c * s + i, 0)),
            pl.BlockSpec((D, E), lambda i: (0, 0)),
        ],
        out_specs=pl.BlockSpec((tb, E), lambda i: (i, 0)),
        out_shape=jax.ShapeDtypeStruct((tc, E), jnp.float32),
    )(x, W)


def _sc_topk_body(aff_hbm, vis_hbm, aux_hbm, mb_hbm, idx_hbm, gate_hbm,
                  aff_v, vis_v, aux_v, mb_v, idxo_v, gateo_v, *, n_tok):
    wid = lax.axis_index("s") * 2 + lax.axis_index("c")
    base = wid * n_tok

    pltpu.sync_copy(aff_hbm.at[pl.ds(base, n_tok), :], aff_v)
    pltpu.sync_copy(vis_hbm.at[pl.ds(base, n_tok)], vis_v)
    pltpu.sync_copy(aux_hbm, aux_v)
    pltpu.sync_copy(mb_hbm, mb_v)

    ii = lax.broadcasted_iota(jnp.int32, (16,), 0)
    lo8 = ii < 8
    shifted = jnp.maximum(ii - 8, 0)

    pre = []
    for c in range(4):
        a_c = aux_v[pl.ds(c * 16, 16)]
        pre.append((a_c + mb_v[0, pl.ds(c * 16, 16)],
                    a_c + mb_v[1, pl.ds(c * 16, 16)]))

    def merge_top8(ak, av, bk, bv):
        ck = jnp.where(lo8, ak, bk.at[shifted].get(mode="promise_in_bounds"))
        cv = jnp.where(lo8, av, bv.at[shifted].get(mode="promise_in_bounds"))
        return plsc.sort_key_val(ck, cv, descending=True)

    def one_token(t):
        t_splat = jnp.full((16,), t, jnp.int32)
        vis_t = plsc.load_gather(vis_v, [t_splat])
        visb = vis_t != 0
        sk, sv = [], []
        for c in range(4):
            b_c = aff_v[t, pl.ds(c * 16, 16)] + jnp.where(visb, pre[c][1], pre[c][0])
            kk, vv = plsc.sort_key_val(b_c, ii + 16 * c, descending=True)
            sk.append(kk)
            sv.append(vv)
        mk0, mv0 = merge_top8(sk[0], sv[0], sk[1], sv[1])
        mk1, mv1 = merge_top8(sk[2], sv[2], sk[3], sv[3])
        _, fv = merge_top8(mk0, mv0, mk1, mv1)

        g_aff = plsc.load_gather(aff_v, [t_splat, fv], mask=lo8)
        g_aff = jnp.where(lo8, g_aff, 0.0)
        gate8 = g_aff / (jnp.sum(g_aff, axis=0) + 1e-12)

        opos = t * 8 + ii
        plsc.store_scatter(idxo_v, [opos], fv, mask=lo8)
        plsc.store_scatter(gateo_v, [opos], gate8, mask=lo8)

    def tok_body(i, _):
        one_token(2 * i)
        one_token(2 * i + 1)
        return _

    lax.fori_loop(0, n_tok // 2, tok_body, 0)

    pltpu.sync_copy(idxo_v, idx_hbm.at[pl.ds(base * 8, n_tok * 8)])
    pltpu.sync_copy(gateo_v, gate_hbm.at[pl.ds(base * 8, n_tok * 8)])


def _sc_topk_call(aff, visi, aux, mb):
    Tc, E = aff.shape
    n_tok = Tc // 32
    body = functools.partial(_sc_topk_body, n_tok=n_tok)
    fn = pl.kernel(
        body,
        out_type=[
            jax.ShapeDtypeStruct((Tc * 8,), jnp.int32),
            jax.ShapeDtypeStruct((Tc * 8,), jnp.float32),
        ],
        mesh=plsc.VectorSubcoreMesh(core_axis_name="c", subcore_axis_name="s"),
        compiler_params=pltpu.CompilerParams(needs_layout_passes=False),
        scratch_types=[
            pltpu.VMEM((n_tok, E), jnp.float32),
            pltpu.VMEM((n_tok,), jnp.int32),
            pltpu.VMEM((E,), jnp.float32),
            pltpu.VMEM((2, E), jnp.float32),
            pltpu.VMEM((n_tok * 8,), jnp.int32),
            pltpu.VMEM((n_tok * 8,), jnp.float32),
        ],
    )
    return fn(aff, visi, aux, mb)


def kernel(x, is_visual, W, aux_free_bias, modality_bias):
    T, D = x.shape
    E = W.shape[1]
    nc = _NCHUNK
    tc = T // nc
    visi = is_visual.astype(jnp.int32)

    affs, idxs, gates = [], [], []
    for c in range(nc):
        aff_c = _affinity_call(x, W, c, nc)
        idxf, gatef = _sc_topk_call(aff_c, visi[c * tc:(c + 1) * tc],
                                    aux_free_bias, modality_bias)
        affs.append(aff_c)
        idxs.append(idxf.reshape(tc, 8))
        gates.append(gatef.reshape(tc, 8))

    return (jnp.concatenate(idxs, axis=0),
            jnp.concatenate(gates, axis=0),
            jnp.concatenate(affs, axis=0))
